# baseline (speedup 1.0000x reference)
import jax
import jax.numpy as jnp
from jax import lax
from jax.experimental import pallas as pl
from jax.experimental.pallas import tpu as pltpu
from jax.experimental.pallas import tpu_sc as plsc

N_NODES = 10000
IN_CH = 128
HID = 64
OUT_CH = 40
NEG = 0.2

NPAD = 10240
NC = 2
NS = 16
NW = NC * NS
CHUNK = 128
EC = 82
EPAD = NW * EC * CHUNK
ROWS_PER_TILE = NPAD // NS
BM = 1280
GRID_M = NPAD // BM


def _k1_body(x_ref, w_ref, av_s_ref, av_d_ref, h_ref, a1_ref, a2_ref):
    m = pl.program_id(0)
    h = jnp.dot(x_ref[...], w_ref[...], preferred_element_type=jnp.float32)
    h_ref[...] = h
    rows = m * BM + lax.broadcasted_iota(jnp.int32, (BM, 1), 0)
    valid = rows < N_NODES
    a_s = jnp.sum(h * av_s_ref[...], axis=1, keepdims=True)
    a_d = jnp.sum(h * av_d_ref[...], axis=1, keepdims=True)
    a1_ref[...] = jnp.where(valid, a_s, -1e30)
    a2_ref[...] = jnp.where(valid, a_d, -1e30)


def _k1(xp, W, av_s, av_d):
    return pl.pallas_call(
        _k1_body,
        grid=(GRID_M,),
        in_specs=[
            pl.BlockSpec((BM, IN_CH), lambda m: (m, 0)),
            pl.BlockSpec((IN_CH, HID), lambda m: (0, 0)),
            pl.BlockSpec((1, HID), lambda m: (0, 0)),
            pl.BlockSpec((1, HID), lambda m: (0, 0)),
        ],
        out_specs=[
            pl.BlockSpec((BM, HID), lambda m: (m, 0)),
            pl.BlockSpec((BM, 1), lambda m: (m, 0)),
            pl.BlockSpec((BM, 1), lambda m: (m, 0)),
        ],
        out_shape=[
            jax.ShapeDtypeStruct((NPAD, HID), jnp.float32),
            jax.ShapeDtypeStruct((NPAD, 1), jnp.float32),
            jax.ShapeDtypeStruct((NPAD, 1), jnp.float32),
        ],
    )(xp, W, av_s, av_d)


def _sc_body(h_hbm, as_hbm, ad_hbm, src_hbm, dst_hbm, outp_hbm, den_hbm,
             as_v, ad_v, den_v, src_v, dst_v, row_a, row_b, acc_sh, gsem):
    cid = lax.axis_index("c")
    sid = lax.axis_index("s")
    wid = cid * NS + sid

    pltpu.sync_copy(as_hbm, as_v)
    pltpu.sync_copy(ad_hbm, ad_v)
    pltpu.sync_copy(src_hbm.at[wid], src_v)
    pltpu.sync_copy(dst_hbm.at[wid], dst_v)

    zeros16 = jnp.zeros((16,), jnp.float32)

    def _zero_den(i, c):
        den_v[pl.ds(i * 16, 16)] = zeros16
        return c

    lax.fori_loop(0, NPAD // 16, _zero_den, 0)

    def _zero_row(r, c):
        for cc in range(HID // 16):
            row_b[r, pl.ds(cc * 16, 16)] = zeros16
        return c

    lax.fori_loop(0, CHUNK, _zero_row, 0)

    def _mx(i, carry):
        ma, md = carry
        ma = jnp.maximum(ma, as_v[pl.ds(i * 16, 16)])
        md = jnp.maximum(md, ad_v[pl.ds(i * 16, 16)])
        return ma, md

    neg = jnp.full((16,), -3e38, jnp.float32)
    ma, md = lax.fori_loop(0, NPAD // 16, _mx, (neg, neg))
    sa = ma[0]
    sd = md[0]
    for l in range(1, 16):
        sa = jnp.maximum(sa, ma[l])
        sd = jnp.maximum(sd, md[l])
    cmax = sa + sd
    cshift = jnp.maximum(cmax, NEG * cmax)

    base = sid * ROWS_PER_TILE
    for t in range(ROWS_PER_TILE // CHUNK):
        pltpu.sync_copy(row_b, acc_sh.at[pl.ds(base + t * CHUNK, CHUNK)])
    pltpu.async_copy(h_hbm.at[src_v.at[0]], row_a, gsem)
    plsc.subcore_barrier()

    bufs = (row_a, row_b)

    def _pair(j2, c):
        for b in range(2):
            j = j2 * 2 + b
            buf = bufs[b]
            exs = []
            for k in range(CHUNK // 16):
                s_idx = src_v[j, pl.ds(k * 16, 16)]
                d_idx = dst_v[j, pl.ds(k * 16, 16)]
                e = (plsc.load_gather(as_v, [s_idx])
                     + plsc.load_gather(ad_v, [d_idx]))
                e = jnp.where(e > 0, e, NEG * e)
                ex = jnp.exp(e - cshift)
                plsc.addupdate_scatter(den_v, [d_idx], ex)
                exs.append(ex)
            pltpu.make_async_copy(h_hbm.at[src_v.at[j]], buf, gsem).wait()
            pltpu.async_copy(h_hbm.at[src_v.at[j + 1]], bufs[1 - b], gsem)
            for k in range(CHUNK // 16):
                for l in range(16):
                    s = exs[k][l]
                    r = k * 16 + l
                    for col in range(HID // 16):
                        buf[r, pl.ds(col * 16, 16)] = buf[r, pl.ds(col * 16, 16)] * s
            pltpu.sync_copy(buf, acc_sh.at[dst_v.at[j]], add=True)
        return c

    lax.fori_loop(0, EC // 2, _pair, 0)

    pltpu.make_async_copy(h_hbm.at[src_v.at[0]], row_a, gsem).wait()

    pltpu.sync_copy(den_v, den_hbm.at[wid])
    plsc.subcore_barrier()
    pltpu.sync_copy(acc_sh.at[pl.ds(base, ROWS_PER_TILE)],
                    outp_hbm.at[cid, pl.ds(base, ROWS_PER_TILE)])


def _sc_gat(h, a_s, a_d, srcb, dstb):
    return pl.kernel(
        _sc_body,
        out_type=(
            jax.ShapeDtypeStruct((NC, NPAD, HID), jnp.float32),
            jax.ShapeDtypeStruct((NW, NPAD), jnp.float32),
        ),
        mesh=plsc.VectorSubcoreMesh(core_axis_name="c", subcore_axis_name="s"),
        compiler_params=pltpu.CompilerParams(
            needs_layout_passes=False, use_tc_tiling_on_sc=False),
        scratch_types=[
            pltpu.VMEM((NPAD,), jnp.float32),
            pltpu.VMEM((NPAD,), jnp.float32),
            pltpu.VMEM((NPAD,), jnp.float32),
            pltpu.VMEM((EC + 1, CHUNK), jnp.int32),
            pltpu.VMEM((EC, CHUNK), jnp.int32),
            pltpu.VMEM((CHUNK, HID), jnp.float32),
            pltpu.VMEM((CHUNK, HID), jnp.float32),
            pltpu.VMEM_SHARED((NPAD, HID), jnp.float32),
            pltpu.SemaphoreType.DMA,
        ],
    )(h, a_s, a_d, srcb, dstb)


def _k2_body(outp_ref, den_ref, b_ref, w2_ref, av_s_ref, av_d_ref,
             h1_ref, h2h_ref, a1_ref, a2_ref):
    m = pl.program_id(0)
    agg = outp_ref[0] + outp_ref[1]
    den = jnp.sum(den_ref[...], axis=1, keepdims=True)
    x2 = agg / (den + 1e-38) + b_ref[...]
    h1 = jnp.where(x2 > 0, x2, jnp.exp(x2) - 1.0)
    h1_ref[...] = h1
    h2h = jnp.dot(h1, w2_ref[...], preferred_element_type=jnp.float32)
    h2h_ref[...] = h2h
    rows = m * BM + lax.broadcasted_iota(jnp.int32, (BM, 1), 0)
    valid = rows < N_NODES
    a_s = jnp.sum(h2h * av_s_ref[...], axis=1, keepdims=True)
    a_d = jnp.sum(h2h * av_d_ref[...], axis=1, keepdims=True)
    a1_ref[...] = jnp.where(valid, a_s, -1e30)
    a2_ref[...] = jnp.where(valid, a_d, -1e30)


def _k2(outp, dent, b, W2, av_s, av_d):
    return pl.pallas_call(
        _k2_body,
        grid=(GRID_M,),
        in_specs=[
            pl.BlockSpec((NC, BM, HID), lambda m: (0, m, 0)),
            pl.BlockSpec((BM, NW), lambda m: (m, 0)),
            pl.BlockSpec((1, HID), lambda m: (0, 0)),
            pl.BlockSpec((HID, HID), lambda m: (0, 0)),
            pl.BlockSpec((1, HID), lambda m: (0, 0)),
            pl.BlockSpec((1, HID), lambda m: (0, 0)),
        ],
        out_specs=[
            pl.BlockSpec((BM, HID), lambda m: (m, 0)),
            pl.BlockSpec((BM, HID), lambda m: (m, 0)),
            pl.BlockSpec((BM, 1), lambda m: (m, 0)),
            pl.BlockSpec((BM, 1), lambda m: (m, 0)),
        ],
        out_shape=[
            jax.ShapeDtypeStruct((NPAD, HID), jnp.float32),
            jax.ShapeDtypeStruct((NPAD, HID), jnp.float32),
            jax.ShapeDtypeStruct((NPAD, 1), jnp.float32),
            jax.ShapeDtypeStruct((NPAD, 1), jnp.float32),
        ],
    )(outp, dent, b, W2, av_s, av_d)


def _k3_body(outp_ref, den_ref, b_ref, h1_ref, w1_ref, w2_ref, lb_ref, o_ref):
    agg = outp_ref[0] + outp_ref[1]
    den = jnp.sum(den_ref[...], axis=1, keepdims=True)
    x2 = agg / (den + 1e-38) + b_ref[...]
    h2 = jnp.where(x2 > 0, x2, jnp.exp(x2) - 1.0)
    logits = (jnp.dot(h1_ref[...], w1_ref[...], preferred_element_type=jnp.float32)
              + jnp.dot(h2, w2_ref[...], preferred_element_type=jnp.float32)
              + lb_ref[...])
    mx = jnp.max(logits, axis=1, keepdims=True)
    sh = logits - mx
    lse = jnp.log(jnp.sum(jnp.exp(sh), axis=1, keepdims=True))
    o_ref[...] = sh - lse


def _k3(outp, dent, b, h1, linW1, linW2, lb):
    return pl.pallas_call(
        _k3_body,
        grid=(GRID_M,),
        in_specs=[
            pl.BlockSpec((NC, BM, HID), lambda m: (0, m, 0)),
            pl.BlockSpec((BM, NW), lambda m: (m, 0)),
            pl.BlockSpec((1, HID), lambda m: (0, 0)),
            pl.BlockSpec((BM, HID), lambda m: (m, 0)),
            pl.BlockSpec((HID, OUT_CH), lambda m: (0, 0)),
            pl.BlockSpec((HID, OUT_CH), lambda m: (0, 0)),
            pl.BlockSpec((1, OUT_CH), lambda m: (0, 0)),
        ],
        out_specs=[pl.BlockSpec((BM, OUT_CH), lambda m: (m, 0))],
        out_shape=[jax.ShapeDtypeStruct((NPAD, OUT_CH), jnp.float32)],
    )(outp, dent, b, h1, linW1, linW2, lb)


def kernel(x, edge_index, W1, a_src1, a_dst1, b1, W2, a_src2, a_dst2, b2,
           linW, linb):
    n = x.shape[0]
    loops = jnp.arange(n, dtype=jnp.int32)
    src = jnp.concatenate([edge_index[0].astype(jnp.int32), loops])
    dst = jnp.concatenate([edge_index[1].astype(jnp.int32), loops])
    padn = EPAD - src.shape[0]
    fill = jnp.full((padn,), N_NODES, jnp.int32)
    srcb = jnp.concatenate([src, fill]).reshape(NW, EC, CHUNK)
    srcb = jnp.pad(srcb, ((0, 0), (0, 1), (0, 0)), constant_values=N_NODES)
    dstb = jnp.concatenate([dst, fill]).reshape(NW, EC, CHUNK)
    xp = jnp.pad(x, ((0, NPAD - n), (0, 0)))

    h1h, a1, a2 = _k1(xp, W1, a_src1.reshape(1, HID), a_dst1.reshape(1, HID))
    outp1, den1 = _sc_gat(h1h, a1.reshape(NPAD), a2.reshape(NPAD), srcb, dstb)
    h1, h2h, a21, a22 = _k2(outp1, den1.T, b1.reshape(1, HID), W2,
                            a_src2.reshape(1, HID), a_dst2.reshape(1, HID))
    outp2, den2 = _sc_gat(h2h, a21.reshape(NPAD), a22.reshape(NPAD), srcb, dstb)
    (out,) = _k3(outp2, den2.T, b2.reshape(1, HID), h1,
                 linW[:HID], linW[HID:], linb.reshape(1, OUT_CH))
    return out[:n]


# speedup vs baseline: 1.3444x; 1.3444x over previous
import jax
import jax.numpy as jnp
from jax import lax
from jax.experimental import pallas as pl
from jax.experimental.pallas import tpu as pltpu
from jax.experimental.pallas import tpu_sc as plsc

N_NODES = 10000
IN_CH = 128
HID = 64
OUT_CH = 40
NEG = 0.2

NPAD = 10240
NC = 2
NS = 16
NW = NC * NS
CHUNK = 128
EC = 81
EPAD = NW * EC * CHUNK
ROWS_PER_TILE = NPAD // NS
BM = 1280
GRID_M = NPAD // BM


def _k1_body(x_ref, w_ref, av_s_ref, av_d_ref, h_ref, a1_ref, a2_ref):
    m = pl.program_id(0)
    h = jnp.dot(x_ref[...], w_ref[...], preferred_element_type=jnp.float32)
    h_ref[...] = h
    rows = m * BM + lax.broadcasted_iota(jnp.int32, (BM, 1), 0)
    valid = rows < N_NODES
    a_s = jnp.sum(h * av_s_ref[...], axis=1, keepdims=True)
    a_d = jnp.sum(h * av_d_ref[...], axis=1, keepdims=True)
    a1_ref[...] = jnp.where(valid, a_s, -1e30)
    a2_ref[...] = jnp.where(valid, a_d, -1e30)


def _k1(xp, W, av_s, av_d):
    return pl.pallas_call(
        _k1_body,
        grid=(GRID_M,),
        in_specs=[
            pl.BlockSpec((BM, IN_CH), lambda m: (m, 0)),
            pl.BlockSpec((IN_CH, HID), lambda m: (0, 0)),
            pl.BlockSpec((1, HID), lambda m: (0, 0)),
            pl.BlockSpec((1, HID), lambda m: (0, 0)),
        ],
        out_specs=[
            pl.BlockSpec((BM, HID), lambda m: (m, 0)),
            pl.BlockSpec((BM, 1), lambda m: (m, 0)),
            pl.BlockSpec((BM, 1), lambda m: (m, 0)),
        ],
        out_shape=[
            jax.ShapeDtypeStruct((NPAD, HID), jnp.float32),
            jax.ShapeDtypeStruct((NPAD, 1), jnp.float32),
            jax.ShapeDtypeStruct((NPAD, 1), jnp.float32),
        ],
    )(xp, W, av_s, av_d)


def _sc_body(h_hbm, as_hbm, ad_hbm, src_hbm, dst_hbm, outp_hbm, den_hbm,
             as_v, ad_v, den_v, src_v, dst_v, row_a, row_b, acc_sh, gsem):
    cid = lax.axis_index("c")
    sid = lax.axis_index("s")
    wid = cid * NS + sid

    pltpu.sync_copy(as_hbm, as_v)
    pltpu.sync_copy(ad_hbm, ad_v)
    pltpu.sync_copy(src_hbm.at[wid], src_v)
    pltpu.sync_copy(dst_hbm.at[wid], dst_v)

    zeros16 = jnp.zeros((16,), jnp.float32)

    def _zero_den(i, c):
        den_v[pl.ds(i * 16, 16)] = zeros16
        return c

    lax.fori_loop(0, NPAD // 16, _zero_den, 0)

    def _zero_row(r, c):
        for cc in range(HID // 16):
            row_b[r, pl.ds(cc * 16, 16)] = zeros16
        return c

    lax.fori_loop(0, CHUNK, _zero_row, 0)

    def _mx(i, carry):
        ma, md = carry
        ma = jnp.maximum(ma, as_v[pl.ds(i * 16, 16)])
        md = jnp.maximum(md, ad_v[pl.ds(i * 16, 16)])
        return ma, md

    neg = jnp.full((16,), -3e38, jnp.float32)
    ma, md = lax.fori_loop(0, NPAD // 16, _mx, (neg, neg))
    sa = ma[0]
    sd = md[0]
    for l in range(1, 16):
        sa = jnp.maximum(sa, ma[l])
        sd = jnp.maximum(sd, md[l])
    cmax = sa + sd
    cshift = jnp.maximum(cmax, NEG * cmax)

    base = sid * ROWS_PER_TILE
    for t in range(ROWS_PER_TILE // CHUNK):
        pltpu.sync_copy(row_b, acc_sh.at[pl.ds(base + t * CHUNK, CHUNK)])
    plsc.subcore_barrier()

    def _chunk(j, c):
        cp = pltpu.async_copy(h_hbm.at[src_v.at[j]], row_a, gsem)
        exs = []
        for k in range(CHUNK // 16):
            s_idx = src_v[j, pl.ds(k * 16, 16)]
            d_idx = dst_v[j, pl.ds(k * 16, 16)]
            e = (plsc.load_gather(as_v, [s_idx])
                 + plsc.load_gather(ad_v, [d_idx]))
            e = jnp.where(e > 0, e, NEG * e)
            ex = jnp.exp(e - cshift)
            plsc.addupdate_scatter(den_v, [d_idx], ex)
            exs.append(ex)
        cp.wait()
        for k in range(CHUNK // 16):
            for l in range(16):
                s = exs[k][l]
                r = k * 16 + l
                for col in range(HID // 16):
                    row_a[r, pl.ds(col * 16, 16)] = row_a[r, pl.ds(col * 16, 16)] * s
        pltpu.sync_copy(row_a, acc_sh.at[dst_v.at[j]], add=True)
        return c

    lax.fori_loop(0, EC, _chunk, 0)

    pltpu.sync_copy(den_v, den_hbm.at[wid])
    plsc.subcore_barrier()
    pltpu.sync_copy(acc_sh.at[pl.ds(base, ROWS_PER_TILE)],
                    outp_hbm.at[cid, pl.ds(base, ROWS_PER_TILE)])


def _sc_gat(h, a_s, a_d, srcb, dstb):
    return pl.kernel(
        _sc_body,
        out_type=(
            jax.ShapeDtypeStruct((NC, NPAD, HID), jnp.float32),
            jax.ShapeDtypeStruct((NW, NPAD), jnp.float32),
        ),
        mesh=plsc.VectorSubcoreMesh(core_axis_name="c", subcore_axis_name="s"),
        compiler_params=pltpu.CompilerParams(
            needs_layout_passes=False, use_tc_tiling_on_sc=False),
        scratch_types=[
            pltpu.VMEM((NPAD,), jnp.float32),
            pltpu.VMEM((NPAD,), jnp.float32),
            pltpu.VMEM((NPAD,), jnp.float32),
            pltpu.VMEM((EC, CHUNK), jnp.int32),
            pltpu.VMEM((EC, CHUNK), jnp.int32),
            pltpu.VMEM((CHUNK, HID), jnp.float32),
            pltpu.VMEM((CHUNK, HID), jnp.float32),
            pltpu.VMEM_SHARED((NPAD, HID), jnp.float32),
            pltpu.SemaphoreType.DMA,
        ],
    )(h, a_s, a_d, srcb, dstb)


def _k2_body(outp_ref, den_ref, b_ref, w2_ref, av_s_ref, av_d_ref,
             h1_ref, h2h_ref, a1_ref, a2_ref):
    m = pl.program_id(0)
    agg = outp_ref[0] + outp_ref[1]
    den = jnp.sum(den_ref[...], axis=1, keepdims=True)
    x2 = agg / (den + 1e-38) + b_ref[...]
    h1 = jnp.where(x2 > 0, x2, jnp.exp(x2) - 1.0)
    h1_ref[...] = h1
    h2h = jnp.dot(h1, w2_ref[...], preferred_element_type=jnp.float32)
    h2h_ref[...] = h2h
    rows = m * BM + lax.broadcasted_iota(jnp.int32, (BM, 1), 0)
    valid = rows < N_NODES
    a_s = jnp.sum(h2h * av_s_ref[...], axis=1, keepdims=True)
    a_d = jnp.sum(h2h * av_d_ref[...], axis=1, keepdims=True)
    a1_ref[...] = jnp.where(valid, a_s, -1e30)
    a2_ref[...] = jnp.where(valid, a_d, -1e30)


def _k2(outp, dent, b, W2, av_s, av_d):
    return pl.pallas_call(
        _k2_body,
        grid=(GRID_M,),
        in_specs=[
            pl.BlockSpec((NC, BM, HID), lambda m: (0, m, 0)),
            pl.BlockSpec((BM, NW), lambda m: (m, 0)),
            pl.BlockSpec((1, HID), lambda m: (0, 0)),
            pl.BlockSpec((HID, HID), lambda m: (0, 0)),
            pl.BlockSpec((1, HID), lambda m: (0, 0)),
            pl.BlockSpec((1, HID), lambda m: (0, 0)),
        ],
        out_specs=[
            pl.BlockSpec((BM, HID), lambda m: (m, 0)),
            pl.BlockSpec((BM, HID), lambda m: (m, 0)),
            pl.BlockSpec((BM, 1), lambda m: (m, 0)),
            pl.BlockSpec((BM, 1), lambda m: (m, 0)),
        ],
        out_shape=[
            jax.ShapeDtypeStruct((NPAD, HID), jnp.float32),
            jax.ShapeDtypeStruct((NPAD, HID), jnp.float32),
            jax.ShapeDtypeStruct((NPAD, 1), jnp.float32),
            jax.ShapeDtypeStruct((NPAD, 1), jnp.float32),
        ],
    )(outp, dent, b, W2, av_s, av_d)


def _k3_body(outp_ref, den_ref, b_ref, h1_ref, w1_ref, w2_ref, lb_ref, o_ref):
    agg = outp_ref[0] + outp_ref[1]
    den = jnp.sum(den_ref[...], axis=1, keepdims=True)
    x2 = agg / (den + 1e-38) + b_ref[...]
    h2 = jnp.where(x2 > 0, x2, jnp.exp(x2) - 1.0)
    logits = (jnp.dot(h1_ref[...], w1_ref[...], preferred_element_type=jnp.float32)
              + jnp.dot(h2, w2_ref[...], preferred_element_type=jnp.float32)
              + lb_ref[...])
    mx = jnp.max(logits, axis=1, keepdims=True)
    sh = logits - mx
    lse = jnp.log(jnp.sum(jnp.exp(sh), axis=1, keepdims=True))
    o_ref[...] = sh - lse


def _k3(outp, dent, b, h1, linW1, linW2, lb):
    return pl.pallas_call(
        _k3_body,
        grid=(GRID_M,),
        in_specs=[
            pl.BlockSpec((NC, BM, HID), lambda m: (0, m, 0)),
            pl.BlockSpec((BM, NW), lambda m: (m, 0)),
            pl.BlockSpec((1, HID), lambda m: (0, 0)),
            pl.BlockSpec((BM, HID), lambda m: (m, 0)),
            pl.BlockSpec((HID, OUT_CH), lambda m: (0, 0)),
            pl.BlockSpec((HID, OUT_CH), lambda m: (0, 0)),
            pl.BlockSpec((1, OUT_CH), lambda m: (0, 0)),
        ],
        out_specs=[pl.BlockSpec((BM, OUT_CH), lambda m: (m, 0))],
        out_shape=[jax.ShapeDtypeStruct((NPAD, OUT_CH), jnp.float32)],
    )(outp, dent, b, h1, linW1, linW2, lb)


def kernel(x, edge_index, W1, a_src1, a_dst1, b1, W2, a_src2, a_dst2, b2,
           linW, linb):
    n = x.shape[0]
    loops = jnp.arange(n, dtype=jnp.int32)
    src = jnp.concatenate([edge_index[0].astype(jnp.int32), loops])
    dst = jnp.concatenate([edge_index[1].astype(jnp.int32), loops])
    padn = EPAD - src.shape[0]
    fill = jnp.full((padn,), N_NODES, jnp.int32)
    srcb = jnp.concatenate([src, fill]).reshape(NW, EC, CHUNK)
    dstb = jnp.concatenate([dst, fill]).reshape(NW, EC, CHUNK)
    xp = jnp.pad(x, ((0, NPAD - n), (0, 0)))

    h1h, a1, a2 = _k1(xp, W1, a_src1.reshape(1, HID), a_dst1.reshape(1, HID))
    outp1, den1 = _sc_gat(h1h, a1.reshape(NPAD), a2.reshape(NPAD), srcb, dstb)
    h1, h2h, a21, a22 = _k2(outp1, den1.T, b1.reshape(1, HID), W2,
                            a_src2.reshape(1, HID), a_dst2.reshape(1, HID))
    outp2, den2 = _sc_gat(h2h, a21.reshape(NPAD), a22.reshape(NPAD), srcb, dstb)
    (out,) = _k3(outp2, den2.T, b2.reshape(1, HID), h1,
                 linW[:HID], linW[HID:], linb.reshape(1, OUT_CH))
    return out[:n]


# speedup vs baseline: 1.5476x; 1.1511x over previous
import jax
import jax.numpy as jnp
from jax import lax
from jax.experimental import pallas as pl
from jax.experimental.pallas import tpu as pltpu
from jax.experimental.pallas import tpu_sc as plsc

N_NODES = 10000
IN_CH = 128
HID = 64
OUT_CH = 40
NEG = 0.2

NPAD = 10240
NC = 2
NS = 16
NW = NC * NS
CHUNK = 128
EC = 81
EPAD = NW * EC * CHUNK
ROWS_PER_TILE = NPAD // NS
BM = 1280
GRID_M = NPAD // BM


def _k1_body(x_ref, w_ref, av_s_ref, av_d_ref, h_ref, a1_ref, a2_ref):
    m = pl.program_id(0)
    h = jnp.dot(x_ref[...], w_ref[...], preferred_element_type=jnp.float32)
    h_ref[...] = h
    rows = m * BM + lax.broadcasted_iota(jnp.int32, (BM, 1), 0)
    valid = rows < N_NODES
    a_s = jnp.sum(h * av_s_ref[...], axis=1, keepdims=True)
    a_d = jnp.sum(h * av_d_ref[...], axis=1, keepdims=True)
    a1_ref[...] = jnp.where(valid, a_s, -1e30)
    a2_ref[...] = jnp.where(valid, a_d, -1e30)


def _k1(xp, W, av_s, av_d):
    return pl.pallas_call(
        _k1_body,
        grid=(GRID_M,),
        in_specs=[
            pl.BlockSpec((BM, IN_CH), lambda m: (m, 0)),
            pl.BlockSpec((IN_CH, HID), lambda m: (0, 0)),
            pl.BlockSpec((1, HID), lambda m: (0, 0)),
            pl.BlockSpec((1, HID), lambda m: (0, 0)),
        ],
        out_specs=[
            pl.BlockSpec((BM, HID), lambda m: (m, 0)),
            pl.BlockSpec((BM, 1), lambda m: (m, 0)),
            pl.BlockSpec((BM, 1), lambda m: (m, 0)),
        ],
        out_shape=[
            jax.ShapeDtypeStruct((NPAD, HID), jnp.float32),
            jax.ShapeDtypeStruct((NPAD, 1), jnp.float32),
            jax.ShapeDtypeStruct((NPAD, 1), jnp.float32),
        ],
    )(xp, W, av_s, av_d)


def _sc_body(h_hbm, as_hbm, ad_hbm, src_hbm, dst_hbm, outp_hbm, den_hbm,
             as_v, ad_v, den_v, src_v, dst_v, row_a, row_b, acc_sh, gsem):
    cid = lax.axis_index("c")
    sid = lax.axis_index("s")
    wid = cid * NS + sid

    pltpu.sync_copy(as_hbm, as_v)
    pltpu.sync_copy(ad_hbm, ad_v)
    pltpu.sync_copy(src_hbm.at[wid], src_v)
    pltpu.sync_copy(dst_hbm.at[wid], dst_v)

    zeros16 = jnp.zeros((16,), jnp.float32)

    def _zero_den(i, c):
        den_v[pl.ds(i * 16, 16)] = zeros16
        return c

    lax.fori_loop(0, NPAD // 16, _zero_den, 0)

    def _zero_row(r, c):
        for cc in range(HID // 16):
            row_b[r, pl.ds(cc * 16, 16)] = zeros16
        return c

    lax.fori_loop(0, CHUNK, _zero_row, 0)

    def _mx(i, carry):
        ma, md = carry
        ma = jnp.maximum(ma, as_v[pl.ds(i * 16, 16)])
        md = jnp.maximum(md, ad_v[pl.ds(i * 16, 16)])
        return ma, md

    neg = jnp.full((16,), -3e38, jnp.float32)
    ma, md = lax.fori_loop(0, NPAD // 16, _mx, (neg, neg))
    sa = ma[0]
    sd = md[0]
    for l in range(1, 16):
        sa = jnp.maximum(sa, ma[l])
        sd = jnp.maximum(sd, md[l])
    cmax = sa + sd
    cshift = jnp.maximum(cmax, NEG * cmax)

    base = sid * ROWS_PER_TILE
    for t in range(ROWS_PER_TILE // CHUNK):
        pltpu.sync_copy(row_b, acc_sh.at[pl.ds(base + t * CHUNK, CHUNK)])
    plsc.subcore_barrier()

    def _chunk(j, c):
        cp = pltpu.async_copy(h_hbm.at[src_v.at[j]], row_a, gsem)
        exs = []
        for k in range(CHUNK // 16):
            s_idx = src_v[j, pl.ds(k * 16, 16)]
            d_idx = dst_v[j, pl.ds(k * 16, 16)]
            e = (plsc.load_gather(as_v, [s_idx])
                 + plsc.load_gather(ad_v, [d_idx]))
            e = jnp.where(e > 0, e, NEG * e)
            ex = jnp.exp(e - cshift)
            plsc.addupdate_scatter(den_v, [d_idx], ex)
            exs.append(ex)
        cp.wait()
        for k in range(CHUNK // 16):
            for l in range(16):
                s = exs[k][l]
                r = k * 16 + l
                for col in range(HID // 16):
                    row_a[r, pl.ds(col * 16, 16)] = row_a[r, pl.ds(col * 16, 16)] * s
        pltpu.sync_copy(row_a, acc_sh.at[dst_v.at[j]], add=True)
        return c

    lax.fori_loop(0, EC, _chunk, 0)

    pltpu.sync_copy(den_v, den_hbm.at[wid])
    plsc.subcore_barrier()
    pltpu.sync_copy(acc_sh.at[pl.ds(base, ROWS_PER_TILE)],
                    outp_hbm.at[cid, pl.ds(base, ROWS_PER_TILE)])


def _sc_gat(h, a_s, a_d, srcb, dstb):
    return pl.kernel(
        _sc_body,
        out_type=(
            jax.ShapeDtypeStruct((NC, NPAD, HID), jnp.float32),
            jax.ShapeDtypeStruct((NW, NPAD), jnp.float32),
        ),
        mesh=plsc.VectorSubcoreMesh(core_axis_name="c", subcore_axis_name="s"),
        compiler_params=pltpu.CompilerParams(
            needs_layout_passes=False, use_tc_tiling_on_sc=False),
        scratch_types=[
            pltpu.VMEM((NPAD,), jnp.float32),
            pltpu.VMEM((NPAD,), jnp.float32),
            pltpu.VMEM((NPAD,), jnp.float32),
            pltpu.VMEM((EC, CHUNK), jnp.int32),
            pltpu.VMEM((EC, CHUNK), jnp.int32),
            pltpu.VMEM((CHUNK, HID), jnp.float32),
            pltpu.VMEM((CHUNK, HID), jnp.float32),
            pltpu.VMEM_SHARED((NPAD, HID), jnp.float32),
            pltpu.SemaphoreType.DMA,
        ],
    )(h, a_s, a_d, srcb, dstb)


def _k2_body(outp_ref, den_ref, b_ref, w2_ref, av_s_ref, av_d_ref,
             h1_ref, h2h_ref, a1_ref, a2_ref):
    m = pl.program_id(0)
    agg = outp_ref[0] + outp_ref[1]
    den = jnp.sum(den_ref[...], axis=1, keepdims=True)
    x2 = agg / (den + 1e-38) + b_ref[...]
    h1 = jnp.where(x2 > 0, x2, jnp.exp(x2) - 1.0)
    h1_ref[...] = h1
    h2h = jnp.dot(h1, w2_ref[...], preferred_element_type=jnp.float32)
    h2h_ref[...] = h2h
    rows = m * BM + lax.broadcasted_iota(jnp.int32, (BM, 1), 0)
    valid = rows < N_NODES
    a_s = jnp.sum(h2h * av_s_ref[...], axis=1, keepdims=True)
    a_d = jnp.sum(h2h * av_d_ref[...], axis=1, keepdims=True)
    a1_ref[...] = jnp.where(valid, a_s, -1e30)
    a2_ref[...] = jnp.where(valid, a_d, -1e30)


def _k2(outp, dent, b, W2, av_s, av_d):
    return pl.pallas_call(
        _k2_body,
        grid=(GRID_M,),
        in_specs=[
            pl.BlockSpec((NC, BM, HID), lambda m: (0, m, 0)),
            pl.BlockSpec((BM, NW), lambda m: (m, 0)),
            pl.BlockSpec((1, HID), lambda m: (0, 0)),
            pl.BlockSpec((HID, HID), lambda m: (0, 0)),
            pl.BlockSpec((1, HID), lambda m: (0, 0)),
            pl.BlockSpec((1, HID), lambda m: (0, 0)),
        ],
        out_specs=[
            pl.BlockSpec((BM, HID), lambda m: (m, 0)),
            pl.BlockSpec((BM, HID), lambda m: (m, 0)),
            pl.BlockSpec((BM, 1), lambda m: (m, 0)),
            pl.BlockSpec((BM, 1), lambda m: (m, 0)),
        ],
        out_shape=[
            jax.ShapeDtypeStruct((NPAD, HID), jnp.float32),
            jax.ShapeDtypeStruct((NPAD, HID), jnp.float32),
            jax.ShapeDtypeStruct((NPAD, 1), jnp.float32),
            jax.ShapeDtypeStruct((NPAD, 1), jnp.float32),
        ],
    )(outp, dent, b, W2, av_s, av_d)


def _k3_body(outp_ref, den_ref, b_ref, h1_ref, w1_ref, w2_ref, lb_ref, o_ref):
    agg = outp_ref[0] + outp_ref[1]
    den = jnp.sum(den_ref[...], axis=1, keepdims=True)
    x2 = agg / (den + 1e-38) + b_ref[...]
    h2 = jnp.where(x2 > 0, x2, jnp.exp(x2) - 1.0)
    logits = (jnp.dot(h1_ref[...], w1_ref[...], preferred_element_type=jnp.float32)
              + jnp.dot(h2, w2_ref[...], preferred_element_type=jnp.float32)
              + lb_ref[...])
    mx = jnp.max(logits, axis=1, keepdims=True)
    sh = logits - mx
    lse = jnp.log(jnp.sum(jnp.exp(sh), axis=1, keepdims=True))
    o_ref[...] = sh - lse


def _k3(outp, dent, b, h1, linW1, linW2, lb):
    return pl.pallas_call(
        _k3_body,
        grid=(GRID_M,),
        in_specs=[
            pl.BlockSpec((NC, BM, HID), lambda m: (0, m, 0)),
            pl.BlockSpec((BM, NW), lambda m: (m, 0)),
            pl.BlockSpec((1, HID), lambda m: (0, 0)),
            pl.BlockSpec((BM, HID), lambda m: (m, 0)),
            pl.BlockSpec((HID, OUT_CH), lambda m: (0, 0)),
            pl.BlockSpec((HID, OUT_CH), lambda m: (0, 0)),
            pl.BlockSpec((1, OUT_CH), lambda m: (0, 0)),
        ],
        out_specs=[pl.BlockSpec((BM, OUT_CH), lambda m: (m, 0))],
        out_shape=[jax.ShapeDtypeStruct((NPAD, OUT_CH), jnp.float32)],
    )(outp, dent, b, h1, linW1, linW2, lb)


def kernel(x, edge_index, W1, a_src1, a_dst1, b1, W2, a_src2, a_dst2, b2,
           linW, linb):
    n = x.shape[0]
    loops = jnp.arange(n, dtype=jnp.int32)
    src = jnp.concatenate([edge_index[0].astype(jnp.int32), loops])
    dst = jnp.concatenate([edge_index[1].astype(jnp.int32), loops])
    padn = EPAD - src.shape[0]
    fill = N_NODES + jnp.arange(padn, dtype=jnp.int32) % (NPAD - N_NODES)
    srcb = jnp.concatenate([src, fill]).reshape(NW, EC, CHUNK)
    dstb = jnp.concatenate([dst, fill]).reshape(NW, EC, CHUNK)
    xp = jnp.pad(x, ((0, NPAD - n), (0, 0)))

    h1h, a1, a2 = _k1(xp, W1, a_src1.reshape(1, HID), a_dst1.reshape(1, HID))
    outp1, den1 = _sc_gat(h1h, a1.reshape(NPAD), a2.reshape(NPAD), srcb, dstb)
    h1, h2h, a21, a22 = _k2(outp1, den1.T, b1.reshape(1, HID), W2,
                            a_src2.reshape(1, HID), a_dst2.reshape(1, HID))
    outp2, den2 = _sc_gat(h2h, a21.reshape(NPAD), a22.reshape(NPAD), srcb, dstb)
    (out,) = _k3(outp2, den2.T, b2.reshape(1, HID), h1,
                 linW[:HID], linW[HID:], linb.reshape(1, OUT_CH))
    return out[:n]
